# Initial kernel scaffold; baseline (speedup 1.0000x reference)
import jax
import jax.numpy as jnp
from jax.experimental import pallas as pl


def kernel(x, pos_table):
    raise NotImplementedError("write your pallas kernel here")


# speedup vs baseline: 2.1014x; 2.1014x over previous
import jax
import jax.numpy as jnp
from jax.experimental import pallas as pl

_BLOCK_ROWS = 256


def _add_kernel(x_ref, p_ref, o_ref):
    o_ref[...] = x_ref[...] + p_ref[...]


def kernel(x, pos_table):
    seq_len, d = x.shape
    grid = (seq_len // _BLOCK_ROWS,)
    spec = pl.BlockSpec((_BLOCK_ROWS, d), lambda i: (i, 0))
    return pl.pallas_call(
        _add_kernel,
        grid=grid,
        in_specs=[spec, spec],
        out_specs=spec,
        out_shape=jax.ShapeDtypeStruct((seq_len, d), x.dtype),
    )(x, pos_table)
